# baseline (speedup 1.0000x reference)
import jax
import jax.numpy as jnp
from jax.experimental import pallas as pl
from jax.experimental.pallas import tpu as pltpu


def _round_up(x, m):
    return (x + m - 1) // m * m


def _pre_kernel(x_ref, w1l_ref, w1r_ref, b1_ref, y_ref, z_ref):
    x = x_ref[...].astype(jnp.bfloat16)
    y_ref[...] = jnp.dot(x, w1l_ref[...],
                         preferred_element_type=jnp.float32).astype(jnp.bfloat16)
    z_ref[...] = (jnp.dot(x, w1r_ref[...], preferred_element_type=jnp.float32)
                  + b1_ref[...])


def _agg1_kernel(a_ref, y_ref, z_ref, w2l_ref, h_ref, p_ref, inv_ref,
                 acc_ref, deg_ref):
    k = pl.program_id(1)

    @pl.when(k == 0)
    def _():
        acc_ref[...] = jnp.zeros_like(acc_ref)
        deg_ref[...] = jnp.zeros_like(deg_ref)

    a = a_ref[...]
    acc_ref[...] += jnp.dot(a, y_ref[...],
                            preferred_element_type=jnp.float32)
    deg_ref[...] += jnp.sum(a.astype(jnp.float32), axis=1, keepdims=True)

    @pl.when(k == pl.num_programs(1) - 1)
    def _():
        inv = 1.0 / jnp.maximum(deg_ref[...], 1.0)
        inv_ref[...] = inv
        h = jnp.maximum(acc_ref[...] * inv + z_ref[...], 0.0)
        h_bf = h.astype(jnp.bfloat16)
        h_ref[...] = h_bf
        p_ref[...] = jnp.dot(h_bf, w2l_ref[...],
                             preferred_element_type=jnp.float32).astype(jnp.bfloat16)


def _agg2_kernel(a_ref, p_ref, h_ref, inv_ref, w2r_ref, b2_ref, o_ref, acc_ref):
    k = pl.program_id(1)

    @pl.when(k == 0)
    def _():
        acc_ref[...] = jnp.zeros_like(acc_ref)

    acc_ref[...] += jnp.dot(a_ref[...], p_ref[...],
                            preferred_element_type=jnp.float32)

    @pl.when(k == pl.num_programs(1) - 1)
    def _():
        o_ref[...] = (acc_ref[...] * inv_ref[...]
                      + jnp.dot(h_ref[...], w2r_ref[...],
                                preferred_element_type=jnp.float32)
                      + b2_ref[...])


def kernel(x, edge_index, w1l, w1r, b1, w2l, w2r, b2):
    n, cin = x.shape
    hid = w1l.shape[1]
    cout = w2l.shape[1]

    cin_p = _round_up(cin, 128)
    hid_p = _round_up(hid, 128)
    cout_p = _round_up(cout, 128)

    tm, tk = 512, 1024
    n_pad = _round_up(n, tk)
    grid = (n_pad // tm, n_pad // tk)

    src, dst = edge_index[0], edge_index[1]
    lin = jnp.sort(dst * n_pad + src)
    a = jnp.zeros((n_pad * n_pad,), jnp.bfloat16).at[lin].add(
        1.0, indices_are_sorted=True).reshape(n_pad, n_pad)

    x_p = jnp.pad(x, ((0, n_pad - n), (0, cin_p - cin)))
    w1l_p = jnp.pad(w1l, ((0, cin_p - cin), (0, hid_p - hid))).astype(jnp.bfloat16)
    w1r_p = jnp.pad(w1r, ((0, cin_p - cin), (0, hid_p - hid))).astype(jnp.bfloat16)
    b1_p = jnp.pad(b1, ((0, 0), (0, hid_p - hid))).astype(jnp.float32)
    w2l_p = jnp.pad(w2l, ((0, hid_p - hid), (0, cout_p - cout))).astype(jnp.bfloat16)
    w2r_p = jnp.pad(w2r, ((0, hid_p - hid), (0, cout_p - cout))).astype(jnp.bfloat16)
    b2_p = jnp.pad(b2, ((0, 0), (0, cout_p - cout))).astype(jnp.float32)

    cparams = pltpu.CompilerParams(
        dimension_semantics=("parallel", "arbitrary"),
        vmem_limit_bytes=64 * 1024 * 1024,
    )
    const = lambda *_: (0, 0)

    y, z = pl.pallas_call(
        _pre_kernel,
        grid=(n_pad // tk,),
        in_specs=[
            pl.BlockSpec((tk, cin_p), lambda i: (i, 0)),
            pl.BlockSpec((cin_p, hid_p), lambda i: (0, 0)),
            pl.BlockSpec((cin_p, hid_p), lambda i: (0, 0)),
            pl.BlockSpec((1, hid_p), lambda i: (0, 0)),
        ],
        out_specs=[
            pl.BlockSpec((tk, hid_p), lambda i: (i, 0)),
            pl.BlockSpec((tk, hid_p), lambda i: (i, 0)),
        ],
        out_shape=[
            jax.ShapeDtypeStruct((n_pad, hid_p), jnp.bfloat16),
            jax.ShapeDtypeStruct((n_pad, hid_p), jnp.float32),
        ],
        compiler_params=pltpu.CompilerParams(
            dimension_semantics=("parallel",),
            vmem_limit_bytes=64 * 1024 * 1024,
        ),
    )(x_p, w1l_p, w1r_p, b1_p)

    h, p, inv_col = pl.pallas_call(
        _agg1_kernel,
        grid=grid,
        in_specs=[
            pl.BlockSpec((tm, tk), lambda i, k: (i, k)),
            pl.BlockSpec((tk, hid_p), lambda i, k: (k, 0)),
            pl.BlockSpec((tm, hid_p), lambda i, k: (i, 0)),
            pl.BlockSpec((hid_p, cout_p), const),
        ],
        out_specs=[
            pl.BlockSpec((tm, hid_p), lambda i, k: (i, 0)),
            pl.BlockSpec((tm, cout_p), lambda i, k: (i, 0)),
            pl.BlockSpec((tm, 1), lambda i, k: (i, 0)),
        ],
        out_shape=[
            jax.ShapeDtypeStruct((n_pad, hid_p), jnp.bfloat16),
            jax.ShapeDtypeStruct((n_pad, cout_p), jnp.bfloat16),
            jax.ShapeDtypeStruct((n_pad, 1), jnp.float32),
        ],
        scratch_shapes=[pltpu.VMEM((tm, hid_p), jnp.float32),
                        pltpu.VMEM((tm, 1), jnp.float32)],
        compiler_params=cparams,
    )(a, y, z, w2l_p)

    out_p = pl.pallas_call(
        _agg2_kernel,
        grid=grid,
        in_specs=[
            pl.BlockSpec((tm, tk), lambda i, k: (i, k)),
            pl.BlockSpec((tk, cout_p), lambda i, k: (k, 0)),
            pl.BlockSpec((tm, hid_p), lambda i, k: (i, 0)),
            pl.BlockSpec((tm, 1), lambda i, k: (i, 0)),
            pl.BlockSpec((hid_p, cout_p), const),
            pl.BlockSpec((1, cout_p), const),
        ],
        out_specs=pl.BlockSpec((tm, cout_p), lambda i, k: (i, 0)),
        out_shape=jax.ShapeDtypeStruct((n_pad, cout_p), jnp.float32),
        scratch_shapes=[pltpu.VMEM((tm, cout_p), jnp.float32)],
        compiler_params=cparams,
    )(a, p, h, inv_col, w2r_p, b2_p)

    return out_p[:n, :cout]


# speedup vs baseline: 1.1299x; 1.1299x over previous
import jax
import jax.numpy as jnp
from jax.experimental import pallas as pl
from jax.experimental.pallas import tpu as pltpu


def _round_up(x, m):
    return (x + m - 1) // m * m


def _pre_kernel(x_ref, w1l_ref, w1r_ref, b1_ref, y_ref, z_ref):
    x = x_ref[...].astype(jnp.bfloat16)
    y_ref[...] = jnp.dot(x, w1l_ref[...],
                         preferred_element_type=jnp.float32).astype(jnp.bfloat16)
    z_ref[...] = (jnp.dot(x, w1r_ref[...], preferred_element_type=jnp.float32)
                  + b1_ref[...])


def _agg1_kernel(a_ref, y_ref, z_ref, w2l_ref, h_ref, p_ref, inv_ref,
                 acc_ref, deg_ref):
    k = pl.program_id(1)

    @pl.when(k == 0)
    def _():
        acc_ref[...] = jnp.zeros_like(acc_ref)
        deg_ref[...] = jnp.zeros_like(deg_ref)

    a = a_ref[...]
    acc_ref[...] += jnp.dot(a, y_ref[...],
                            preferred_element_type=jnp.float32)
    deg_ref[...] += jnp.sum(a.astype(jnp.float32), axis=1, keepdims=True)

    @pl.when(k == pl.num_programs(1) - 1)
    def _():
        inv = 1.0 / jnp.maximum(deg_ref[...], 1.0)
        inv_ref[...] = inv
        h = jnp.maximum(acc_ref[...] * inv + z_ref[...], 0.0)
        h_bf = h.astype(jnp.bfloat16)
        h_ref[...] = h_bf
        p_ref[...] = jnp.dot(h_bf, w2l_ref[...],
                             preferred_element_type=jnp.float32).astype(jnp.bfloat16)


def _agg2_kernel(a_ref, p_ref, h_ref, inv_ref, w2r_ref, b2_ref, o_ref, acc_ref):
    k = pl.program_id(1)

    @pl.when(k == 0)
    def _():
        acc_ref[...] = jnp.zeros_like(acc_ref)

    acc_ref[...] += jnp.dot(a_ref[...], p_ref[...],
                            preferred_element_type=jnp.float32)

    @pl.when(k == pl.num_programs(1) - 1)
    def _():
        o_ref[...] = (acc_ref[...] * inv_ref[...]
                      + jnp.dot(h_ref[...], w2r_ref[...],
                                preferred_element_type=jnp.float32)
                      + b2_ref[...])


def kernel(x, edge_index, w1l, w1r, b1, w2l, w2r, b2):
    n, cin = x.shape
    hid = w1l.shape[1]
    cout = w2l.shape[1]

    cin_p = _round_up(cin, 128)
    hid_p = _round_up(hid, 128)
    cout_p = _round_up(cout, 128)

    tm, tk = 1024, 2048
    n_pad = _round_up(n, tk)
    grid = (n_pad // tm, n_pad // tk)

    src, dst = edge_index[0], edge_index[1]
    lin = dst * n_pad + src
    a = jnp.zeros((n_pad * n_pad,), jnp.bfloat16).at[lin].add(
        1.0).reshape(n_pad, n_pad)

    x_p = jnp.pad(x, ((0, n_pad - n), (0, cin_p - cin)))
    w1l_p = jnp.pad(w1l, ((0, cin_p - cin), (0, hid_p - hid))).astype(jnp.bfloat16)
    w1r_p = jnp.pad(w1r, ((0, cin_p - cin), (0, hid_p - hid))).astype(jnp.bfloat16)
    b1_p = jnp.pad(b1, ((0, 0), (0, hid_p - hid))).astype(jnp.float32)
    w2l_p = jnp.pad(w2l, ((0, hid_p - hid), (0, cout_p - cout))).astype(jnp.bfloat16)
    w2r_p = jnp.pad(w2r, ((0, hid_p - hid), (0, cout_p - cout))).astype(jnp.bfloat16)
    b2_p = jnp.pad(b2, ((0, 0), (0, cout_p - cout))).astype(jnp.float32)

    cparams = pltpu.CompilerParams(
        dimension_semantics=("parallel", "arbitrary"),
        vmem_limit_bytes=64 * 1024 * 1024,
    )
    const = lambda *_: (0, 0)

    y, z = pl.pallas_call(
        _pre_kernel,
        grid=(n_pad // tk,),
        in_specs=[
            pl.BlockSpec((tk, cin_p), lambda i: (i, 0)),
            pl.BlockSpec((cin_p, hid_p), lambda i: (0, 0)),
            pl.BlockSpec((cin_p, hid_p), lambda i: (0, 0)),
            pl.BlockSpec((1, hid_p), lambda i: (0, 0)),
        ],
        out_specs=[
            pl.BlockSpec((tk, hid_p), lambda i: (i, 0)),
            pl.BlockSpec((tk, hid_p), lambda i: (i, 0)),
        ],
        out_shape=[
            jax.ShapeDtypeStruct((n_pad, hid_p), jnp.bfloat16),
            jax.ShapeDtypeStruct((n_pad, hid_p), jnp.float32),
        ],
        compiler_params=pltpu.CompilerParams(
            dimension_semantics=("parallel",),
            vmem_limit_bytes=64 * 1024 * 1024,
        ),
    )(x_p, w1l_p, w1r_p, b1_p)

    h, p, inv_col = pl.pallas_call(
        _agg1_kernel,
        grid=grid,
        in_specs=[
            pl.BlockSpec((tm, tk), lambda i, k: (i, k)),
            pl.BlockSpec((tk, hid_p), lambda i, k: (k, 0)),
            pl.BlockSpec((tm, hid_p), lambda i, k: (i, 0)),
            pl.BlockSpec((hid_p, cout_p), const),
        ],
        out_specs=[
            pl.BlockSpec((tm, hid_p), lambda i, k: (i, 0)),
            pl.BlockSpec((tm, cout_p), lambda i, k: (i, 0)),
            pl.BlockSpec((tm, 1), lambda i, k: (i, 0)),
        ],
        out_shape=[
            jax.ShapeDtypeStruct((n_pad, hid_p), jnp.bfloat16),
            jax.ShapeDtypeStruct((n_pad, cout_p), jnp.bfloat16),
            jax.ShapeDtypeStruct((n_pad, 1), jnp.float32),
        ],
        scratch_shapes=[pltpu.VMEM((tm, hid_p), jnp.float32),
                        pltpu.VMEM((tm, 1), jnp.float32)],
        compiler_params=cparams,
    )(a, y, z, w2l_p)

    out_p = pl.pallas_call(
        _agg2_kernel,
        grid=grid,
        in_specs=[
            pl.BlockSpec((tm, tk), lambda i, k: (i, k)),
            pl.BlockSpec((tk, cout_p), lambda i, k: (k, 0)),
            pl.BlockSpec((tm, hid_p), lambda i, k: (i, 0)),
            pl.BlockSpec((tm, 1), lambda i, k: (i, 0)),
            pl.BlockSpec((hid_p, cout_p), const),
            pl.BlockSpec((1, cout_p), const),
        ],
        out_specs=pl.BlockSpec((tm, cout_p), lambda i, k: (i, 0)),
        out_shape=jax.ShapeDtypeStruct((n_pad, cout_p), jnp.float32),
        scratch_shapes=[pltpu.VMEM((tm, cout_p), jnp.float32)],
        compiler_params=cparams,
    )(a, p, h, inv_col, w2r_p, b2_p)

    return out_p[:n, :cout]


# speedup vs baseline: 1.1588x; 1.0256x over previous
import jax
import jax.numpy as jnp
from jax.experimental import pallas as pl
from jax.experimental.pallas import tpu as pltpu


def _round_up(x, m):
    return (x + m - 1) // m * m


def _pre_kernel(x_ref, w1l_ref, w1r_ref, b1_ref, y_ref, z_ref):
    x = x_ref[...].astype(jnp.bfloat16)
    y_ref[...] = jnp.dot(x, w1l_ref[...],
                         preferred_element_type=jnp.float32).astype(jnp.bfloat16)
    z_ref[...] = (jnp.dot(x, w1r_ref[...], preferred_element_type=jnp.float32)
                  + b1_ref[...])


def _agg1_kernel(a_ref, y_ref, z_ref, w2l_ref, h_ref, p_ref, inv_ref,
                 acc_ref, deg_ref):
    k = pl.program_id(1)

    @pl.when(k == 0)
    def _():
        acc_ref[...] = jnp.zeros_like(acc_ref)
        deg_ref[...] = jnp.zeros_like(deg_ref)

    a = a_ref[...]
    acc_ref[...] += jnp.dot(a, y_ref[...],
                            preferred_element_type=jnp.float32)
    deg_ref[...] += jnp.sum(a.astype(jnp.float32), axis=1, keepdims=True)

    @pl.when(k == pl.num_programs(1) - 1)
    def _():
        inv = 1.0 / jnp.maximum(deg_ref[...], 1.0)
        inv_ref[...] = inv
        h = jnp.maximum(acc_ref[...] * inv + z_ref[...], 0.0)
        h_bf = h.astype(jnp.bfloat16)
        h_ref[...] = h_bf
        p_ref[...] = jnp.dot(h_bf, w2l_ref[...],
                             preferred_element_type=jnp.float32).astype(jnp.bfloat16)


def _agg2_kernel(a_ref, p_ref, h_ref, inv_ref, w2r_ref, b2_ref, o_ref, acc_ref):
    k = pl.program_id(1)

    @pl.when(k == 0)
    def _():
        acc_ref[...] = jnp.zeros_like(acc_ref)

    acc_ref[...] += jnp.dot(a_ref[...], p_ref[...],
                            preferred_element_type=jnp.float32)

    @pl.when(k == pl.num_programs(1) - 1)
    def _():
        o_ref[...] = (acc_ref[...] * inv_ref[...]
                      + jnp.dot(h_ref[...], w2r_ref[...],
                                preferred_element_type=jnp.float32)
                      + b2_ref[...])


def kernel(x, edge_index, w1l, w1r, b1, w2l, w2r, b2):
    n, cin = x.shape
    hid = w1l.shape[1]
    cout = w2l.shape[1]

    cin_p = _round_up(cin, 128)
    hid_p = _round_up(hid, 128)
    cout_p = _round_up(cout, 128)

    tm, tk = 2048, 2048
    n_pad = _round_up(n, tk)
    grid = (n_pad // tm, n_pad // tk)

    src, dst = edge_index[0], edge_index[1]
    lin = dst * n_pad + src
    a = jnp.zeros((n_pad * n_pad,), jnp.bfloat16).at[lin].add(
        1.0).reshape(n_pad, n_pad)

    x_p = jnp.pad(x, ((0, n_pad - n), (0, cin_p - cin)))
    w1l_p = jnp.pad(w1l, ((0, cin_p - cin), (0, hid_p - hid))).astype(jnp.bfloat16)
    w1r_p = jnp.pad(w1r, ((0, cin_p - cin), (0, hid_p - hid))).astype(jnp.bfloat16)
    b1_p = jnp.pad(b1, ((0, 0), (0, hid_p - hid))).astype(jnp.float32)
    w2l_p = jnp.pad(w2l, ((0, hid_p - hid), (0, cout_p - cout))).astype(jnp.bfloat16)
    w2r_p = jnp.pad(w2r, ((0, hid_p - hid), (0, cout_p - cout))).astype(jnp.bfloat16)
    b2_p = jnp.pad(b2, ((0, 0), (0, cout_p - cout))).astype(jnp.float32)

    cparams = pltpu.CompilerParams(
        dimension_semantics=("parallel", "arbitrary"),
        vmem_limit_bytes=64 * 1024 * 1024,
    )
    const = lambda *_: (0, 0)

    y, z = pl.pallas_call(
        _pre_kernel,
        grid=(n_pad // tk,),
        in_specs=[
            pl.BlockSpec((tk, cin_p), lambda i: (i, 0)),
            pl.BlockSpec((cin_p, hid_p), lambda i: (0, 0)),
            pl.BlockSpec((cin_p, hid_p), lambda i: (0, 0)),
            pl.BlockSpec((1, hid_p), lambda i: (0, 0)),
        ],
        out_specs=[
            pl.BlockSpec((tk, hid_p), lambda i: (i, 0)),
            pl.BlockSpec((tk, hid_p), lambda i: (i, 0)),
        ],
        out_shape=[
            jax.ShapeDtypeStruct((n_pad, hid_p), jnp.bfloat16),
            jax.ShapeDtypeStruct((n_pad, hid_p), jnp.float32),
        ],
        compiler_params=pltpu.CompilerParams(
            dimension_semantics=("parallel",),
            vmem_limit_bytes=64 * 1024 * 1024,
        ),
    )(x_p, w1l_p, w1r_p, b1_p)

    h, p, inv_col = pl.pallas_call(
        _agg1_kernel,
        grid=grid,
        in_specs=[
            pl.BlockSpec((tm, tk), lambda i, k: (i, k)),
            pl.BlockSpec((tk, hid_p), lambda i, k: (k, 0)),
            pl.BlockSpec((tm, hid_p), lambda i, k: (i, 0)),
            pl.BlockSpec((hid_p, cout_p), const),
        ],
        out_specs=[
            pl.BlockSpec((tm, hid_p), lambda i, k: (i, 0)),
            pl.BlockSpec((tm, cout_p), lambda i, k: (i, 0)),
            pl.BlockSpec((tm, 1), lambda i, k: (i, 0)),
        ],
        out_shape=[
            jax.ShapeDtypeStruct((n_pad, hid_p), jnp.bfloat16),
            jax.ShapeDtypeStruct((n_pad, cout_p), jnp.bfloat16),
            jax.ShapeDtypeStruct((n_pad, 1), jnp.float32),
        ],
        scratch_shapes=[pltpu.VMEM((tm, hid_p), jnp.float32),
                        pltpu.VMEM((tm, 1), jnp.float32)],
        compiler_params=cparams,
    )(a, y, z, w2l_p)

    out_p = pl.pallas_call(
        _agg2_kernel,
        grid=grid,
        in_specs=[
            pl.BlockSpec((tm, tk), lambda i, k: (i, k)),
            pl.BlockSpec((tk, cout_p), lambda i, k: (k, 0)),
            pl.BlockSpec((tm, hid_p), lambda i, k: (i, 0)),
            pl.BlockSpec((tm, 1), lambda i, k: (i, 0)),
            pl.BlockSpec((hid_p, cout_p), const),
            pl.BlockSpec((1, cout_p), const),
        ],
        out_specs=pl.BlockSpec((tm, cout_p), lambda i, k: (i, 0)),
        out_shape=jax.ShapeDtypeStruct((n_pad, cout_p), jnp.float32),
        scratch_shapes=[pltpu.VMEM((tm, cout_p), jnp.float32)],
        compiler_params=cparams,
    )(a, p, h, inv_col, w2r_p, b2_p)

    return out_p[:n, :cout]
